# baseline (speedup 1.0000x reference)
import functools

import jax
import jax.numpy as jnp
from jax import lax
from jax.experimental import pallas as pl
from jax.experimental.pallas import tpu as pltpu
from jax.experimental.pallas import tpu_sc as plsc

NC = 2
NS = 16
NW = NC * NS
L = 16
D = 128
NG = 64
K_E = 128
S_CH = 40
_HI = jax.lax.Precision.HIGHEST


@functools.lru_cache(maxsize=None)
def _sc_scatter(tn: int, n_pad: int, e: int):
    k = K_E
    assert e % (NW * 2 * k) == 0
    e_per_w = e // NW
    n_chunks = e_per_w // k
    s = min(S_CH, n_chunks)
    assert n_chunks % s == 0 and s % 2 == 0
    n_stages = n_chunks // s
    n_pairs = s // 2
    rows_per_tile = n_pad // NS
    assert rows_per_tile % k == 0

    mesh = plsc.VectorSubcoreMesh(
        core_axis_name="c", subcore_axis_name="s",
        num_cores=NC, num_subcores=NS)

    @functools.partial(
        pl.kernel, mesh=mesh,
        out_type=jax.ShapeDtypeStruct((NC, n_pad, D), jnp.float32),
        scratch_types=[
            pltpu.VMEM((s, k), jnp.int32),
            pltpu.VMEM((s, k), jnp.int32),
            pltpu.VMEM((s, k), jnp.float32),
            pltpu.VMEM((k, D), jnp.float32),
            pltpu.VMEM((k, D), jnp.float32),
            pltpu.VMEM_SHARED((n_pad, D), jnp.float32),
            pltpu.SemaphoreType.DMA,
            pltpu.SemaphoreType.DMA,
        ],
    )
    def kern(y_hbm, src_hbm, dst_hbm, ew_hbm, out_hbm,
             src_v, dst_v, ew_v, rows0, rows1, z_sh, sem0, sem1):
        cid = lax.axis_index("c")
        sid = lax.axis_index("s")
        wid = sid * NC + cid

        zvec = jnp.zeros((L,), jnp.float32)

        def zrow(r, _):
            for f in range(D // L):
                rows0[r, pl.ds(f * L, L)] = zvec
            return 0
        lax.fori_loop(0, k, zrow, 0)

        def zcopy(t, _):
            pltpu.sync_copy(
                rows0, z_sh.at[pl.ds(sid * rows_per_tile + t * k, k)])
            return 0
        lax.fori_loop(0, rows_per_tile // k, zcopy, 0)
        plsc.subcore_barrier()

        def issue(i, buf, sem):
            pltpu.async_copy(y_hbm.at[src_v.at[i]], buf, sem)

        def wait(buf, sem):
            pltpu.make_async_copy(y_hbm.at[pl.ds(0, k)], buf, sem).wait()

        def scale(buf, i):
            def group(g, _):
                w16 = ew_v[i, pl.ds(g * L, L)]
                for j in range(L):
                    e0 = g * L + j
                    wj = w16[j]
                    for f in range(D // L):
                        sl = pl.ds(f * L, L)
                        buf[e0, sl] = buf[e0, sl] * wj
                return 0
            lax.fori_loop(0, k // L, group, 0)

        def scatter(buf, i):
            pltpu.sync_copy(buf, z_sh.at[dst_v.at[i]], add=True)

        def stage(h, _):
            pltpu.sync_copy(src_hbm.at[wid, pl.ds(h * s, s)], src_v)
            pltpu.sync_copy(dst_hbm.at[wid, pl.ds(h * s, s)], dst_v)
            pltpu.sync_copy(ew_hbm.at[wid, pl.ds(h * s, s)], ew_v)

            issue(0, rows0, sem0)
            issue(1, rows1, sem1)

            def pair(p, _):
                i0 = 2 * p
                wait(rows0, sem0)
                scale(rows0, i0)
                scatter(rows0, i0)

                @pl.when(p + 1 < n_pairs)
                def _():
                    issue(i0 + 2, rows0, sem0)

                wait(rows1, sem1)
                scale(rows1, i0 + 1)
                scatter(rows1, i0 + 1)

                @pl.when(p + 1 < n_pairs)
                def _():
                    issue(i0 + 3, rows1, sem1)
                return 0
            lax.fori_loop(0, n_pairs, pair, 0)
            return 0
        lax.fori_loop(0, n_stages, stage, 0)
        plsc.subcore_barrier()

        pltpu.sync_copy(
            z_sh.at[pl.ds(sid * rows_per_tile, rows_per_tile)],
            out_hbm.at[cid, pl.ds(sid * rows_per_tile, rows_per_tile)])

    return kern


def _tc_layer_body(z_ref, w_ref, b_ref, h_ref):
    agg = z_ref[0] + z_ref[1]
    h_ref[...] = jax.nn.relu(
        jnp.dot(agg, w_ref[...], preferred_element_type=jnp.float32)
        + b_ref[...])


def _pool_head(z_ref, w2_ref, b2_ref, batch_ref, wc_ref, bc_ref):
    n = batch_ref.shape[1]
    agg = z_ref[0, :n] + z_ref[1, :n]
    h = jax.nn.relu(
        jnp.dot(agg, w2_ref[...], preferred_element_type=jnp.float32)
        + b2_ref[...])
    gids = lax.broadcasted_iota(jnp.int32, (NG, n), 0)
    mask = (gids == batch_ref[...]).astype(jnp.float32)
    sums = jnp.dot(mask, h, preferred_element_type=jnp.float32,
                   precision=_HI)
    cnt = jnp.sum(mask, axis=1, keepdims=True)
    pooled = sums / jnp.maximum(cnt, 1.0)
    return jnp.dot(pooled, wc_ref[...],
                   preferred_element_type=jnp.float32) + bc_ref[...]


def _tc_out_body(zw_ref, w2w_ref, b2w_ref, batchw_ref, wcw_ref, bcw_ref,
                 zs_ref, w2s_ref, b2s_ref, batchs_ref, wcs_ref, bcs_ref,
                 wf_ref, bf_ref, out_ref):
    xw = _pool_head(zw_ref, w2w_ref, b2w_ref, batchw_ref, wcw_ref, bcw_ref)
    xs = _pool_head(zs_ref, w2s_ref, b2s_ref, batchs_ref, wcs_ref, bcs_ref)
    out_ref[...] = jnp.dot(xw + xs, wf_ref[...],
                           preferred_element_type=jnp.float32) + bf_ref[...]


def kernel(word_x, word_edge_index, word_batch, word_edge_weight,
           sent_x, sent_edge_index, sent_batch, sent_edge_weight,
           W1w, b1w, W2w, b2w, Wcw, bcw,
           W1s, b1s, W2s, b2s, Wcs, bcs, Wf, bf):
    n_w, e_w = word_x.shape[0], word_edge_weight.shape[0]
    n_s, e_s = sent_x.shape[0], sent_edge_weight.shape[0]

    src_w, dst_w = word_edge_index[0], word_edge_index[1]
    src_s, dst_s = sent_edge_index[0], sent_edge_index[1]

    def _pad_rows(n):
        q = NS * 128
        return ((n + q - 1) // q) * q

    def _pad_edges(src, dst, ew):
        e = src.shape[0]
        q = NW * 2 * K_E
        e_pad = ((e + q - 1) // q) * q
        if e_pad != e:
            pad = e_pad - e
            zi = jnp.zeros((pad,), jnp.int32)
            src = jnp.concatenate([src, zi])
            dst = jnp.concatenate([dst, zi])
            ew = jnp.concatenate([ew, jnp.zeros((pad,), jnp.float32)])
        nc = e_pad // (NW * K_E)
        shp = (NW, nc, K_E)
        return src.reshape(shp), dst.reshape(shp), ew.reshape(shp), e_pad

    np_w, np_s = _pad_rows(n_w), _pad_rows(n_s)
    src_w, dst_w, ew_w, e_w = _pad_edges(src_w, dst_w, word_edge_weight)
    src_s, dst_s, ew_s, e_s = _pad_edges(src_s, dst_s, sent_edge_weight)

    z1w = _sc_scatter(n_w, np_w, e_w)(word_x, src_w, dst_w, ew_w)
    z1s = _sc_scatter(n_s, np_s, e_s)(sent_x, src_s, dst_s, ew_s)

    lay_w = pl.pallas_call(
        _tc_layer_body, out_shape=jax.ShapeDtypeStruct((np_w, D), jnp.float32))
    lay_s = pl.pallas_call(
        _tc_layer_body, out_shape=jax.ShapeDtypeStruct((np_s, D), jnp.float32))
    h1w = lay_w(z1w, W1w, b1w.reshape(1, D))
    h1s = lay_s(z1s, W1s, b1s.reshape(1, D))

    z2w = _sc_scatter(np_w, np_w, e_w)(h1w, src_w, dst_w, ew_w)
    z2s = _sc_scatter(np_s, np_s, e_s)(h1s, src_s, dst_s, ew_s)

    n_cls = Wf.shape[0]
    out = pl.pallas_call(
        _tc_out_body,
        out_shape=jax.ShapeDtypeStruct((NG, n_cls), jnp.float32),
    )(z2w, W2w, b2w.reshape(1, D), word_batch.reshape(1, n_w), Wcw,
      bcw.reshape(1, n_cls),
      z2s, W2s, b2s.reshape(1, D), sent_batch.reshape(1, n_s), Wcs,
      bcs.reshape(1, n_cls), Wf, bf.reshape(1, n_cls))
    return out
